# Initial kernel scaffold; baseline (speedup 1.0000x reference)
import jax
import jax.numpy as jnp
from jax.experimental import pallas as pl


def kernel(x, edge_index, edge_attr, We1, be1, W1a, b1a, W1b, b1b, gamma1, beta1, We2, be2, W2a, b2a, W2b, b2b, gamma2, beta2, Wc, bc, Wr, br, logit_bias):
    raise NotImplementedError("write your pallas kernel here")


# speedup vs baseline: 1.6241x; 1.6241x over previous
import functools

import jax
import jax.numpy as jnp
from jax import lax
from jax.experimental import pallas as pl
from jax.experimental.pallas import tpu as pltpu
from jax.experimental.pallas import tpu_sc as plsc

N = 10000
E = 160000
IN_DIM = 256
HIDDEN = 512
NUM_CLASSES = 4

NC = 2
NS = 16
LANES = 16
CHUNK = 128
K = 128
KH = 64
EPT = E // NS
NSTEP = pl.cdiv(EPT, K)
EPTP = NSTEP * K
EPAD = EPTP - EPT
AGGR_ROWS = N + 8
STRIPE = 640
ZCOP = 40


def _sc_aggregate(xflat, eflat, srcs, dsts, nchunks):
    npass = nchunks // NC
    mesh = plsc.VectorSubcoreMesh(core_axis_name="c", subcore_axis_name="s")
    idx_rows = pl.cdiv(NSTEP, LANES) * LANES

    @functools.partial(
        pl.kernel,
        mesh=mesh,
        out_type=jax.ShapeDtypeStruct((nchunks * N, CHUNK), jnp.float32),
        scratch_types=[
            pltpu.VMEM((1, idx_rows), jnp.int32),
            pltpu.VMEM((idx_rows, K), jnp.int32),
            pltpu.VMEM((idx_rows, K), jnp.int32),
            pltpu.VMEM((idx_rows, K), jnp.int32),
            pltpu.VMEM((KH, CHUNK), jnp.float32),
            pltpu.VMEM((KH, CHUNK), jnp.float32),
            pltpu.VMEM_SHARED((AGGR_ROWS, CHUNK), jnp.float32),
            pltpu.SemaphoreType.DMA,
        ],
    )
    def k(x_hbm, e_hbm, srcs_hbm, dsts_hbm, out_hbm,
          ridx_v, src_v, dst_v, eid_v, xb, eb, aggr, sem):
        c = lax.axis_index("c")
        s = lax.axis_index("s")
        row0 = s * STRIPE
        ncop = jnp.where(s == NS - 1, (N - (NS - 1) * STRIPE) // ZCOP,
                         STRIPE // ZCOP)
        lane = lax.iota(jnp.int32, LANES)

        for j in range(idx_rows // LANES):
            ridx_v[0, pl.ds(j * LANES, LANES)] = jnp.minimum(
                lane + (s * NSTEP + j * LANES), NS * NSTEP - 1)

        pltpu.async_copy(srcs_hbm.at[ridx_v.at[0]], src_v, sem).wait()
        pltpu.async_copy(dsts_hbm.at[ridx_v.at[0]], dst_v, sem).wait()

        def src_off(off):
            def orow(t, carry):
                for j in range(K // LANES):
                    sl = pl.ds(j * LANES, LANES)
                    src_v[t, sl] = src_v[t, sl] + off
                return carry
            lax.fori_loop(0, NSTEP, orow, 0)

        src_off(c * N)

        def zrow(r, carry):
            for j in range(CHUNK // LANES):
                xb[r, pl.ds(j * LANES, LANES)] = jnp.zeros((LANES,), jnp.float32)
            return carry

        lax.fori_loop(0, KH, zrow, 0)

        for p in range(npass):
            f = p * NC + c

            def zcopy(i, carry):
                pltpu.sync_copy(xb.at[pl.ds(0, ZCOP)],
                                aggr.at[pl.ds(row0 + i * ZCOP, ZCOP)])
                return carry

            lax.fori_loop(0, ncop, zcopy, 0)
            plsc.subcore_barrier()

            if p > 0:
                src_off(NC * N)

            ebase = s * EPT

            def erow(t, carry):
                for j in range(K // LANES):
                    sl = pl.ds(j * LANES, LANES)
                    eid_v[t, sl] = f * E + jnp.minimum(
                        lane + (ebase + t * K + j * LANES), E - 1)
                return carry

            lax.fori_loop(0, NSTEP, erow, 0)

            def step(t, carry):
                for h in range(K // KH):
                    hs = pl.ds(h * KH, KH)
                    pltpu.async_copy(x_hbm.at[src_v.at[t, hs]], xb, sem).wait()
                    pltpu.async_copy(e_hbm.at[eid_v.at[t, hs]], eb, sem).wait()

                    def crow(r, carry2):
                        for j in range(CHUNK // LANES):
                            sl = pl.ds(j * LANES, LANES)
                            xb[r, sl] = jnp.maximum(xb[r, sl] + eb[r, sl], 0.0)
                        return carry2

                    lax.fori_loop(0, KH, crow, 0)
                    pltpu.sync_copy(xb, aggr.at[dst_v.at[t, hs]], add=True)
                return carry

            lax.fori_loop(0, NSTEP, step, 0)
            plsc.subcore_barrier()

            def ocopy(i, carry):
                r0 = row0 + i * ZCOP
                pltpu.sync_copy(aggr.at[pl.ds(r0, ZCOP)],
                                out_hbm.at[pl.ds(f * N + r0, ZCOP)])
                return carry

            lax.fori_loop(0, ncop, ocopy, 0)

            if p + 1 < npass:
                lax.fori_loop(0, KH, zrow, 0)

    return k(xflat, eflat, srcs, dsts)


_BE = 2000
_BN = 1000


def _edge_mm(edge_attr, We, be, nchunks):
    eb2 = be.reshape(nchunks, 1, CHUNK)

    def body(a_ref, w_ref, b_ref, o_ref):
        o_ref[...] = (
            jnp.dot(a_ref[...], w_ref[...], preferred_element_type=jnp.float32)
            + b_ref[0]
        )

    grid = (nchunks, E // _BE)
    return pl.pallas_call(
        body,
        grid=grid,
        in_specs=[
            pl.BlockSpec((_BE, edge_attr.shape[1]), lambda f, i: (i, 0)),
            pl.BlockSpec((edge_attr.shape[1], CHUNK), lambda f, i: (0, f)),
            pl.BlockSpec((1, 1, CHUNK), lambda f, i: (f, 0, 0)),
        ],
        out_specs=pl.BlockSpec((_BE, CHUNK), lambda f, i: (f * (E // _BE) + i, 0)),
        out_shape=jax.ShapeDtypeStruct((nchunks * E, CHUNK), jnp.float32),
    )(edge_attr, We, eb2)


def _mlp_stats(h, aggr, Wa, ba, Wb, bb, nchunks):
    in_dim = h.shape[1]

    def body(h_ref, a_ref, wa_ref, ba_ref, wb_ref, bb_ref, u_ref, st_ref):
        i = pl.program_id(0)
        hin = h_ref[...] + jnp.concatenate(
            [a_ref[j] for j in range(nchunks)], axis=-1)
        t = jnp.maximum(
            jnp.dot(hin, wa_ref[...], preferred_element_type=jnp.float32)
            + ba_ref[...], 0.0)
        u = jnp.dot(t, wb_ref[...], preferred_element_type=jnp.float32) + bb_ref[...]
        u_ref[...] = u

        @pl.when(i == 0)
        def _():
            st_ref[...] = jnp.zeros_like(st_ref)

        st_ref[0:1, :] += jnp.sum(u, axis=0, keepdims=True)
        st_ref[1:2, :] += jnp.sum(u * u, axis=0, keepdims=True)

    return pl.pallas_call(
        body,
        grid=(N // _BN,),
        in_specs=[
            pl.BlockSpec((_BN, in_dim), lambda i: (i, 0)),
            pl.BlockSpec((nchunks, _BN, CHUNK), lambda i: (0, i, 0)),
            pl.BlockSpec((in_dim, HIDDEN), lambda i: (0, 0)),
            pl.BlockSpec((1, HIDDEN), lambda i: (0, 0)),
            pl.BlockSpec((HIDDEN, HIDDEN), lambda i: (0, 0)),
            pl.BlockSpec((1, HIDDEN), lambda i: (0, 0)),
        ],
        out_specs=[
            pl.BlockSpec((_BN, HIDDEN), lambda i: (i, 0)),
            pl.BlockSpec((8, HIDDEN), lambda i: (0, 0)),
        ],
        out_shape=[
            jax.ShapeDtypeStruct((N, HIDDEN), jnp.float32),
            jax.ShapeDtypeStruct((8, HIDDEN), jnp.float32),
        ],
    )(h, aggr, Wa, ba.reshape(1, HIDDEN), Wb, bb.reshape(1, HIDDEN))


def _bn_relu(u, stats, gamma, beta):
    def body(u_ref, st_ref, g_ref, b_ref, o_ref):
        mean = st_ref[0:1, :] * (1.0 / N)
        var = st_ref[1:2, :] * (1.0 / N) - mean * mean
        rstd = lax.rsqrt(var + 1e-5)
        o_ref[...] = jnp.maximum(
            (u_ref[...] - mean) * rstd * g_ref[...] + b_ref[...], 0.0)

    return pl.pallas_call(
        body,
        grid=(N // _BN,),
        in_specs=[
            pl.BlockSpec((_BN, HIDDEN), lambda i: (i, 0)),
            pl.BlockSpec((8, HIDDEN), lambda i: (0, 0)),
            pl.BlockSpec((1, HIDDEN), lambda i: (0, 0)),
            pl.BlockSpec((1, HIDDEN), lambda i: (0, 0)),
        ],
        out_specs=pl.BlockSpec((_BN, HIDDEN), lambda i: (i, 0)),
        out_shape=jax.ShapeDtypeStruct((N, HIDDEN), jnp.float32),
    )(u, stats, gamma.reshape(1, HIDDEN), beta.reshape(1, HIDDEN))


def _bn_relu_heads(u, stats, gamma, beta, Wcat, bcat):
    nh = Wcat.shape[1]

    def body(u_ref, st_ref, g_ref, b_ref, w_ref, hb_ref, lo_ref, re_ref):
        mean = st_ref[0:1, :] * (1.0 / N)
        var = st_ref[1:2, :] * (1.0 / N) - mean * mean
        rstd = lax.rsqrt(var + 1e-5)
        h = jnp.maximum(
            (u_ref[...] - mean) * rstd * g_ref[...] + b_ref[...], 0.0)
        o = jnp.dot(h, w_ref[...], preferred_element_type=jnp.float32) + hb_ref[...]
        lo_ref[...] = o[:, :NUM_CLASSES]
        re_ref[...] = o[:, NUM_CLASSES:NUM_CLASSES + 1]

    return pl.pallas_call(
        body,
        grid=(N // _BN,),
        in_specs=[
            pl.BlockSpec((_BN, HIDDEN), lambda i: (i, 0)),
            pl.BlockSpec((8, HIDDEN), lambda i: (0, 0)),
            pl.BlockSpec((1, HIDDEN), lambda i: (0, 0)),
            pl.BlockSpec((1, HIDDEN), lambda i: (0, 0)),
            pl.BlockSpec((HIDDEN, nh), lambda i: (0, 0)),
            pl.BlockSpec((1, nh), lambda i: (0, 0)),
        ],
        out_specs=[
            pl.BlockSpec((_BN, NUM_CLASSES), lambda i: (i, 0)),
            pl.BlockSpec((_BN, 1), lambda i: (i, 0)),
        ],
        out_shape=[
            jax.ShapeDtypeStruct((N, NUM_CLASSES), jnp.float32),
            jax.ShapeDtypeStruct((N, 1), jnp.float32),
        ],
    )(u, stats, gamma.reshape(1, HIDDEN), beta.reshape(1, HIDDEN), Wcat, bcat)


def _chunk_major(h, nchunks):
    return jnp.transpose(h.reshape(N, nchunks, CHUNK), (1, 0, 2)).reshape(
        nchunks * N, CHUNK)


def kernel(x, edge_index, edge_attr, We1, be1, W1a, b1a, W1b, b1b, gamma1,
           beta1, We2, be2, W2a, b2a, W2b, b2b, gamma2, beta2, Wc, bc, Wr, br,
           logit_bias):
    src = edge_index[0]
    dst = edge_index[1]

    src_t = jnp.pad(src.reshape(NS, EPT), ((0, 0), (0, EPAD))
                    ).reshape(NS * NSTEP, K)
    dst_t = jnp.pad(dst.reshape(NS, EPT), ((0, 0), (0, EPAD)),
                    constant_values=N).reshape(NS * NSTEP, K)

    e1 = _edge_mm(edge_attr, We1, be1, nchunks=2)
    xflat1 = _chunk_major(x, 2)
    a1 = _sc_aggregate(xflat1, e1, src_t, dst_t, nchunks=2)
    u1, st1 = _mlp_stats(x, a1.reshape(2, N, CHUNK), W1a, b1a, W1b, b1b, 2)
    h1 = _bn_relu(u1, st1, gamma1, beta1)

    e2 = _edge_mm(edge_attr, We2, be2, nchunks=4)
    xflat2 = _chunk_major(h1, 4)
    a2 = _sc_aggregate(xflat2, e2, src_t, dst_t, nchunks=4)
    u2, st2 = _mlp_stats(h1, a2.reshape(4, N, CHUNK), W2a, b2a, W2b, b2b, 4)

    Wcat = jnp.concatenate([Wc, Wr], axis=1)
    bcat = jnp.concatenate([bc + logit_bias, br]).reshape(1, NUM_CLASSES + 1)
    logits, reg = _bn_relu_heads(u2, st2, gamma2, beta2, Wcat, bcat)
    return (logits, reg[:, 0])


# speedup vs baseline: 1.9155x; 1.1795x over previous
import functools

import jax
import jax.numpy as jnp
from jax import lax
from jax.experimental import pallas as pl
from jax.experimental.pallas import tpu as pltpu
from jax.experimental.pallas import tpu_sc as plsc

N = 10000
E = 160000
IN_DIM = 256
HIDDEN = 512
NUM_CLASSES = 4

NC = 2
NS = 16
LANES = 16
CHUNK = 128
K = 128
KH = 64
EPT = E // NS
NSTEP = pl.cdiv(EPT, K)
EPTP = NSTEP * K
EPAD = EPTP - EPT
AGGR_ROWS = N + 8
STRIPE = 640
ZCOP = 40


def _sc_aggregate(xflat, eflat, srcs, dsts, nchunks):
    npass = nchunks // NC
    mesh = plsc.VectorSubcoreMesh(core_axis_name="c", subcore_axis_name="s")
    idx_rows = pl.cdiv(NSTEP, LANES) * LANES

    @functools.partial(
        pl.kernel,
        mesh=mesh,
        out_type=jax.ShapeDtypeStruct((nchunks * N, CHUNK), jnp.float32),
        scratch_types=[
            pltpu.VMEM((1, idx_rows), jnp.int32),
            pltpu.VMEM((idx_rows, K), jnp.int32),
            pltpu.VMEM((idx_rows, K), jnp.int32),
            pltpu.VMEM((idx_rows, K), jnp.int32),
            pltpu.VMEM((KH, CHUNK), jnp.float32),
            pltpu.VMEM((KH, CHUNK), jnp.float32),
            pltpu.VMEM_SHARED((AGGR_ROWS, CHUNK), jnp.float32),
            pltpu.SemaphoreType.DMA,
        ],
    )
    def k(x_hbm, e_hbm, srcs_hbm, dsts_hbm, out_hbm,
          ridx_v, src_v, dst_v, eid_v, xb, eb, aggr, sem):
        c = lax.axis_index("c")
        s = lax.axis_index("s")
        row0 = s * STRIPE
        ncop = jnp.where(s == NS - 1, (N - (NS - 1) * STRIPE) // ZCOP,
                         STRIPE // ZCOP)
        lane = lax.iota(jnp.int32, LANES)

        for j in range(idx_rows // LANES):
            ridx_v[0, pl.ds(j * LANES, LANES)] = jnp.minimum(
                lane + (s * NSTEP + j * LANES), NS * NSTEP - 1)

        pltpu.async_copy(srcs_hbm.at[ridx_v.at[0]], src_v, sem).wait()
        pltpu.async_copy(dsts_hbm.at[ridx_v.at[0]], dst_v, sem).wait()

        def src_off(off):
            def orow(t, carry):
                for j in range(K // LANES):
                    sl = pl.ds(j * LANES, LANES)
                    src_v[t, sl] = src_v[t, sl] + off
                return carry
            lax.fori_loop(0, NSTEP, orow, 0)

        src_off(c * N)

        def zrow(r, carry):
            for j in range(CHUNK // LANES):
                xb[r, pl.ds(j * LANES, LANES)] = jnp.zeros((LANES,), jnp.float32)
            return carry

        lax.fori_loop(0, KH, zrow, 0)

        for p in range(npass):
            f = p * NC + c

            def zcopy(i, carry):
                pltpu.sync_copy(xb.at[pl.ds(0, ZCOP)],
                                aggr.at[pl.ds(row0 + i * ZCOP, ZCOP)])
                return carry

            lax.fori_loop(0, ncop, zcopy, 0)
            plsc.subcore_barrier()

            if p > 0:
                src_off(NC * N)

            ebase = s * EPT

            def erow(t, carry):
                for j in range(K // LANES):
                    sl = pl.ds(j * LANES, LANES)
                    eid_v[t, sl] = f * E + jnp.minimum(
                        lane + (ebase + t * K + j * LANES), E - 1)
                return carry

            lax.fori_loop(0, NSTEP, erow, 0)

            def step(t, carry):
                for h in range(K // KH):
                    hs = pl.ds(h * KH, KH)
                    cx = pltpu.async_copy(x_hbm.at[src_v.at[t, hs]], xb, sem)
                    ce = pltpu.async_copy(e_hbm.at[eid_v.at[t, hs]], eb, sem)
                    cx.wait()
                    ce.wait()

                    def crow(r, carry2):
                        for j in range(CHUNK // LANES):
                            sl = pl.ds(j * LANES, LANES)
                            xb[r, sl] = jnp.maximum(xb[r, sl] + eb[r, sl], 0.0)
                        return carry2

                    lax.fori_loop(0, KH, crow, 0)
                    pltpu.sync_copy(xb, aggr.at[dst_v.at[t, hs]], add=True)
                return carry

            lax.fori_loop(0, NSTEP, step, 0)
            plsc.subcore_barrier()

            def ocopy(i, carry):
                r0 = row0 + i * ZCOP
                pltpu.sync_copy(aggr.at[pl.ds(r0, ZCOP)],
                                out_hbm.at[pl.ds(f * N + r0, ZCOP)])
                return carry

            lax.fori_loop(0, ncop, ocopy, 0)

            if p + 1 < npass:
                lax.fori_loop(0, KH, zrow, 0)

    return k(xflat, eflat, srcs, dsts)


_BE = 2000
_BN = 1000


def _edge_mm(edge_attr, We, be, nchunks):
    eb2 = be.reshape(nchunks, 1, CHUNK)

    def body(a_ref, w_ref, b_ref, o_ref):
        o_ref[...] = (
            jnp.dot(a_ref[...], w_ref[...], preferred_element_type=jnp.float32)
            + b_ref[0]
        )

    grid = (nchunks, E // _BE)
    return pl.pallas_call(
        body,
        grid=grid,
        in_specs=[
            pl.BlockSpec((_BE, edge_attr.shape[1]), lambda f, i: (i, 0)),
            pl.BlockSpec((edge_attr.shape[1], CHUNK), lambda f, i: (0, f)),
            pl.BlockSpec((1, 1, CHUNK), lambda f, i: (f, 0, 0)),
        ],
        out_specs=pl.BlockSpec((_BE, CHUNK), lambda f, i: (f * (E // _BE) + i, 0)),
        out_shape=jax.ShapeDtypeStruct((nchunks * E, CHUNK), jnp.float32),
    )(edge_attr, We, eb2)


def _mlp_stats(h, aggr, Wa, ba, Wb, bb, nchunks):
    in_dim = h.shape[1]

    def body(h_ref, a_ref, wa_ref, ba_ref, wb_ref, bb_ref, u_ref, st_ref):
        i = pl.program_id(0)
        hin = h_ref[...] + jnp.concatenate(
            [a_ref[j] for j in range(nchunks)], axis=-1)
        t = jnp.maximum(
            jnp.dot(hin, wa_ref[...], preferred_element_type=jnp.float32)
            + ba_ref[...], 0.0)
        u = jnp.dot(t, wb_ref[...], preferred_element_type=jnp.float32) + bb_ref[...]
        u_ref[...] = u

        @pl.when(i == 0)
        def _():
            st_ref[...] = jnp.zeros_like(st_ref)

        st_ref[0:1, :] += jnp.sum(u, axis=0, keepdims=True)
        st_ref[1:2, :] += jnp.sum(u * u, axis=0, keepdims=True)

    return pl.pallas_call(
        body,
        grid=(N // _BN,),
        in_specs=[
            pl.BlockSpec((_BN, in_dim), lambda i: (i, 0)),
            pl.BlockSpec((nchunks, _BN, CHUNK), lambda i: (0, i, 0)),
            pl.BlockSpec((in_dim, HIDDEN), lambda i: (0, 0)),
            pl.BlockSpec((1, HIDDEN), lambda i: (0, 0)),
            pl.BlockSpec((HIDDEN, HIDDEN), lambda i: (0, 0)),
            pl.BlockSpec((1, HIDDEN), lambda i: (0, 0)),
        ],
        out_specs=[
            pl.BlockSpec((_BN, HIDDEN), lambda i: (i, 0)),
            pl.BlockSpec((8, HIDDEN), lambda i: (0, 0)),
        ],
        out_shape=[
            jax.ShapeDtypeStruct((N, HIDDEN), jnp.float32),
            jax.ShapeDtypeStruct((8, HIDDEN), jnp.float32),
        ],
    )(h, aggr, Wa, ba.reshape(1, HIDDEN), Wb, bb.reshape(1, HIDDEN))


def _bn_relu(u, stats, gamma, beta):
    def body(u_ref, st_ref, g_ref, b_ref, o_ref):
        mean = st_ref[0:1, :] * (1.0 / N)
        var = st_ref[1:2, :] * (1.0 / N) - mean * mean
        rstd = lax.rsqrt(var + 1e-5)
        o_ref[...] = jnp.maximum(
            (u_ref[...] - mean) * rstd * g_ref[...] + b_ref[...], 0.0)

    return pl.pallas_call(
        body,
        grid=(N // _BN,),
        in_specs=[
            pl.BlockSpec((_BN, HIDDEN), lambda i: (i, 0)),
            pl.BlockSpec((8, HIDDEN), lambda i: (0, 0)),
            pl.BlockSpec((1, HIDDEN), lambda i: (0, 0)),
            pl.BlockSpec((1, HIDDEN), lambda i: (0, 0)),
        ],
        out_specs=pl.BlockSpec((_BN, HIDDEN), lambda i: (i, 0)),
        out_shape=jax.ShapeDtypeStruct((N, HIDDEN), jnp.float32),
    )(u, stats, gamma.reshape(1, HIDDEN), beta.reshape(1, HIDDEN))


def _bn_relu_heads(u, stats, gamma, beta, Wcat, bcat):
    nh = Wcat.shape[1]

    def body(u_ref, st_ref, g_ref, b_ref, w_ref, hb_ref, lo_ref, re_ref):
        mean = st_ref[0:1, :] * (1.0 / N)
        var = st_ref[1:2, :] * (1.0 / N) - mean * mean
        rstd = lax.rsqrt(var + 1e-5)
        h = jnp.maximum(
            (u_ref[...] - mean) * rstd * g_ref[...] + b_ref[...], 0.0)
        o = jnp.dot(h, w_ref[...], preferred_element_type=jnp.float32) + hb_ref[...]
        lo_ref[...] = o[:, :NUM_CLASSES]
        re_ref[...] = o[:, NUM_CLASSES:NUM_CLASSES + 1]

    return pl.pallas_call(
        body,
        grid=(N // _BN,),
        in_specs=[
            pl.BlockSpec((_BN, HIDDEN), lambda i: (i, 0)),
            pl.BlockSpec((8, HIDDEN), lambda i: (0, 0)),
            pl.BlockSpec((1, HIDDEN), lambda i: (0, 0)),
            pl.BlockSpec((1, HIDDEN), lambda i: (0, 0)),
            pl.BlockSpec((HIDDEN, nh), lambda i: (0, 0)),
            pl.BlockSpec((1, nh), lambda i: (0, 0)),
        ],
        out_specs=[
            pl.BlockSpec((_BN, NUM_CLASSES), lambda i: (i, 0)),
            pl.BlockSpec((_BN, 1), lambda i: (i, 0)),
        ],
        out_shape=[
            jax.ShapeDtypeStruct((N, NUM_CLASSES), jnp.float32),
            jax.ShapeDtypeStruct((N, 1), jnp.float32),
        ],
    )(u, stats, gamma.reshape(1, HIDDEN), beta.reshape(1, HIDDEN), Wcat, bcat)


def _chunk_major(h, nchunks):
    return jnp.transpose(h.reshape(N, nchunks, CHUNK), (1, 0, 2)).reshape(
        nchunks * N, CHUNK)


def kernel(x, edge_index, edge_attr, We1, be1, W1a, b1a, W1b, b1b, gamma1,
           beta1, We2, be2, W2a, b2a, W2b, b2b, gamma2, beta2, Wc, bc, Wr, br,
           logit_bias):
    src = edge_index[0]
    dst = edge_index[1]

    src_t = jnp.pad(src.reshape(NS, EPT), ((0, 0), (0, EPAD))
                    ).reshape(NS * NSTEP, K)
    dst_t = jnp.pad(dst.reshape(NS, EPT), ((0, 0), (0, EPAD)),
                    constant_values=N).reshape(NS * NSTEP, K)

    e1 = _edge_mm(edge_attr, We1, be1, nchunks=2)
    xflat1 = _chunk_major(x, 2)
    a1 = _sc_aggregate(xflat1, e1, src_t, dst_t, nchunks=2)
    u1, st1 = _mlp_stats(x, a1.reshape(2, N, CHUNK), W1a, b1a, W1b, b1b, 2)
    h1 = _bn_relu(u1, st1, gamma1, beta1)

    e2 = _edge_mm(edge_attr, We2, be2, nchunks=4)
    xflat2 = _chunk_major(h1, 4)
    a2 = _sc_aggregate(xflat2, e2, src_t, dst_t, nchunks=4)
    u2, st2 = _mlp_stats(h1, a2.reshape(4, N, CHUNK), W2a, b2a, W2b, b2b, 4)

    Wcat = jnp.concatenate([Wc, Wr], axis=1)
    bcat = jnp.concatenate([bc + logit_bias, br]).reshape(1, NUM_CLASSES + 1)
    logits, reg = _bn_relu_heads(u2, st2, gamma2, beta2, Wcat, bcat)
    return (logits, reg[:, 0])


# speedup vs baseline: 2.2290x; 1.1636x over previous
import functools

import jax
import jax.numpy as jnp
from jax import lax
from jax.experimental import pallas as pl
from jax.experimental.pallas import tpu as pltpu
from jax.experimental.pallas import tpu_sc as plsc

N = 10000
E = 160000
IN_DIM = 256
HIDDEN = 512
NUM_CLASSES = 4

NC = 2
NS = 16
LANES = 16
CHUNK = 128
K = 128
KH = 64
EPT = E // NS
NSTEP = pl.cdiv(EPT, K)
EPTP = NSTEP * K
EPAD = EPTP - EPT
AGGR_ROWS = N + 8
STRIPE = 640
ZCOP = 40


def _sc_aggregate(xflat, eflat, sds, nchunks):
    npass = nchunks // NC
    mesh = plsc.VectorSubcoreMesh(core_axis_name="c", subcore_axis_name="s")
    idx_rows = pl.cdiv(NSTEP, LANES) * LANES

    @functools.partial(
        pl.kernel,
        mesh=mesh,
        out_type=jax.ShapeDtypeStruct((nchunks * N, CHUNK), jnp.float32),
        scratch_types=[
            pltpu.VMEM((1, idx_rows), jnp.int32),
            pltpu.VMEM((idx_rows, K), jnp.int32),
            pltpu.VMEM((1, KH), jnp.int32),
            pltpu.VMEM((1, KH), jnp.int32),
            pltpu.VMEM((1, KH), jnp.int32),
            pltpu.VMEM((1, KH), jnp.int32),
            pltpu.VMEM((1, KH), jnp.int32),
            pltpu.VMEM((1, KH), jnp.int32),
            pltpu.VMEM((KH, CHUNK), jnp.float32),
            pltpu.VMEM((KH, CHUNK), jnp.float32),
            pltpu.VMEM((KH, CHUNK), jnp.float32),
            pltpu.VMEM((KH, CHUNK), jnp.float32),
            pltpu.VMEM_SHARED((AGGR_ROWS, CHUNK), jnp.float32),
            pltpu.SemaphoreType.DMA,
        ],
    )
    def k(x_hbm, e_hbm, sds_hbm, out_hbm, ridx_v, sd_v,
          gsrc0, gsrc1, gdst0, gdst1, geid0, geid1,
          xb0, xb1, eb0, eb1, aggr, sem):
        gsrc = (gsrc0, gsrc1)
        gdst = (gdst0, gdst1)
        geid = (geid0, geid1)
        xb = (xb0, xb1)
        eb = (eb0, eb1)
        c = lax.axis_index("c")
        s = lax.axis_index("s")
        row0 = s * STRIPE
        ncop = jnp.where(s == NS - 1, (N - (NS - 1) * STRIPE) // ZCOP,
                         STRIPE // ZCOP)
        lane = lax.iota(jnp.int32, LANES)
        ebase = s * EPT

        for j in range(idx_rows // LANES):
            ridx_v[0, pl.ds(j * LANES, LANES)] = jnp.minimum(
                lane + (s * NSTEP + j * LANES), NS * NSTEP - 1)

        pltpu.async_copy(sds_hbm.at[ridx_v.at[0]], sd_v, sem).wait()

        def build_lists(i, f, slot):
            t = jnp.minimum(lax.shift_right_logical(i, 1), NSTEP - 1)
            h = i - 2 * lax.shift_right_logical(i, 1)
            for j in range(KH // LANES):
                sl = pl.ds(j * LANES, LANES)
                v = sd_v[t, pl.ds(h * KH + j * LANES, LANES)]
                q = lax.shift_right_logical(v, 16)
                gsrc[slot][0, sl] = q + f * N
                gdst[slot][0, sl] = v - q * 65536
                geid[slot][0, sl] = f * E + jnp.minimum(
                    lane + (ebase + i * KH + j * LANES), E - 1)

        def issue(slot):
            pltpu.async_copy(e_hbm.at[geid[slot].at[0]], eb[slot], sem)
            pltpu.async_copy(x_hbm.at[gsrc[slot].at[0]], xb[slot], sem)

        def drain(slot):
            pltpu.make_async_copy(e_hbm.at[geid[slot].at[0]], eb[slot],
                                  sem).wait()
            pltpu.make_async_copy(x_hbm.at[gsrc[slot].at[0]], xb[slot],
                                  sem).wait()

        def zrow(r, carry):
            for j in range(CHUNK // LANES):
                xb0[r, pl.ds(j * LANES, LANES)] = jnp.zeros((LANES,),
                                                            jnp.float32)
            return carry

        for p in range(npass):
            f = p * NC + c

            lax.fori_loop(0, ZCOP, zrow, 0)

            def zcopy(i, carry):
                pltpu.sync_copy(xb0.at[pl.ds(0, ZCOP)],
                                aggr.at[pl.ds(row0 + i * ZCOP, ZCOP)])
                return carry

            lax.fori_loop(0, ncop, zcopy, 0)
            plsc.subcore_barrier()

            build_lists(jnp.int32(0), f, 0)
            issue(0)

            def step(t, carry):
                for h in range(2):
                    i = 2 * t + h
                    b = h
                    nb = 1 - h
                    drain(b)
                    build_lists(i + 1, f, nb)
                    issue(nb)

                    def crow(r, carry2):
                        for j in range(CHUNK // LANES):
                            sl = pl.ds(j * LANES, LANES)
                            xb[b][r, sl] = jnp.maximum(
                                xb[b][r, sl] + eb[b][r, sl], 0.0)
                        return carry2

                    lax.fori_loop(0, KH, crow, 0)
                    pltpu.sync_copy(xb[b], aggr.at[gdst[b].at[0]], add=True)
                return carry

            lax.fori_loop(0, NSTEP, step, 0)
            drain(0)
            plsc.subcore_barrier()

            def ocopy(i, carry):
                r0 = row0 + i * ZCOP
                pltpu.sync_copy(aggr.at[pl.ds(r0, ZCOP)],
                                out_hbm.at[pl.ds(f * N + r0, ZCOP)])
                return carry

            lax.fori_loop(0, ncop, ocopy, 0)

    return k(xflat, eflat, sds)


_BE = 2000
_BN = 1000


def _edge_mm(edge_attr, We, be, nchunks):
    eb2 = be.reshape(nchunks, 1, CHUNK)

    def body(a_ref, w_ref, b_ref, o_ref):
        o_ref[...] = (
            jnp.dot(a_ref[...], w_ref[...], preferred_element_type=jnp.float32)
            + b_ref[0]
        )

    grid = (nchunks, E // _BE)
    return pl.pallas_call(
        body,
        grid=grid,
        in_specs=[
            pl.BlockSpec((_BE, edge_attr.shape[1]), lambda f, i: (i, 0)),
            pl.BlockSpec((edge_attr.shape[1], CHUNK), lambda f, i: (0, f)),
            pl.BlockSpec((1, 1, CHUNK), lambda f, i: (f, 0, 0)),
        ],
        out_specs=pl.BlockSpec((_BE, CHUNK), lambda f, i: (f * (E // _BE) + i, 0)),
        out_shape=jax.ShapeDtypeStruct((nchunks * E, CHUNK), jnp.float32),
    )(edge_attr, We, eb2)


def _mlp_stats(h, aggr, Wa, ba, Wb, bb, nchunks):
    in_dim = h.shape[1]

    def body(h_ref, a_ref, wa_ref, ba_ref, wb_ref, bb_ref, u_ref, st_ref):
        i = pl.program_id(0)
        hin = h_ref[...] + jnp.concatenate(
            [a_ref[j] for j in range(nchunks)], axis=-1)
        t = jnp.maximum(
            jnp.dot(hin, wa_ref[...], preferred_element_type=jnp.float32)
            + ba_ref[...], 0.0)
        u = jnp.dot(t, wb_ref[...], preferred_element_type=jnp.float32) + bb_ref[...]
        u_ref[...] = u

        @pl.when(i == 0)
        def _():
            st_ref[...] = jnp.zeros_like(st_ref)

        st_ref[0:1, :] += jnp.sum(u, axis=0, keepdims=True)
        st_ref[1:2, :] += jnp.sum(u * u, axis=0, keepdims=True)

    return pl.pallas_call(
        body,
        grid=(N // _BN,),
        in_specs=[
            pl.BlockSpec((_BN, in_dim), lambda i: (i, 0)),
            pl.BlockSpec((nchunks, _BN, CHUNK), lambda i: (0, i, 0)),
            pl.BlockSpec((in_dim, HIDDEN), lambda i: (0, 0)),
            pl.BlockSpec((1, HIDDEN), lambda i: (0, 0)),
            pl.BlockSpec((HIDDEN, HIDDEN), lambda i: (0, 0)),
            pl.BlockSpec((1, HIDDEN), lambda i: (0, 0)),
        ],
        out_specs=[
            pl.BlockSpec((_BN, HIDDEN), lambda i: (i, 0)),
            pl.BlockSpec((8, HIDDEN), lambda i: (0, 0)),
        ],
        out_shape=[
            jax.ShapeDtypeStruct((N, HIDDEN), jnp.float32),
            jax.ShapeDtypeStruct((8, HIDDEN), jnp.float32),
        ],
    )(h, aggr, Wa, ba.reshape(1, HIDDEN), Wb, bb.reshape(1, HIDDEN))


def _bn_relu(u, stats, gamma, beta):
    def body(u_ref, st_ref, g_ref, b_ref, o_ref):
        mean = st_ref[0:1, :] * (1.0 / N)
        var = st_ref[1:2, :] * (1.0 / N) - mean * mean
        rstd = lax.rsqrt(var + 1e-5)
        o_ref[...] = jnp.maximum(
            (u_ref[...] - mean) * rstd * g_ref[...] + b_ref[...], 0.0)

    return pl.pallas_call(
        body,
        grid=(N // _BN,),
        in_specs=[
            pl.BlockSpec((_BN, HIDDEN), lambda i: (i, 0)),
            pl.BlockSpec((8, HIDDEN), lambda i: (0, 0)),
            pl.BlockSpec((1, HIDDEN), lambda i: (0, 0)),
            pl.BlockSpec((1, HIDDEN), lambda i: (0, 0)),
        ],
        out_specs=pl.BlockSpec((_BN, HIDDEN), lambda i: (i, 0)),
        out_shape=jax.ShapeDtypeStruct((N, HIDDEN), jnp.float32),
    )(u, stats, gamma.reshape(1, HIDDEN), beta.reshape(1, HIDDEN))


def _bn_relu_heads(u, stats, gamma, beta, Wcat, bcat):
    nh = Wcat.shape[1]

    def body(u_ref, st_ref, g_ref, b_ref, w_ref, hb_ref, lo_ref, re_ref):
        mean = st_ref[0:1, :] * (1.0 / N)
        var = st_ref[1:2, :] * (1.0 / N) - mean * mean
        rstd = lax.rsqrt(var + 1e-5)
        h = jnp.maximum(
            (u_ref[...] - mean) * rstd * g_ref[...] + b_ref[...], 0.0)
        o = jnp.dot(h, w_ref[...], preferred_element_type=jnp.float32) + hb_ref[...]
        lo_ref[...] = o[:, :NUM_CLASSES]
        re_ref[...] = o[:, NUM_CLASSES:NUM_CLASSES + 1]

    return pl.pallas_call(
        body,
        grid=(N // _BN,),
        in_specs=[
            pl.BlockSpec((_BN, HIDDEN), lambda i: (i, 0)),
            pl.BlockSpec((8, HIDDEN), lambda i: (0, 0)),
            pl.BlockSpec((1, HIDDEN), lambda i: (0, 0)),
            pl.BlockSpec((1, HIDDEN), lambda i: (0, 0)),
            pl.BlockSpec((HIDDEN, nh), lambda i: (0, 0)),
            pl.BlockSpec((1, nh), lambda i: (0, 0)),
        ],
        out_specs=[
            pl.BlockSpec((_BN, NUM_CLASSES), lambda i: (i, 0)),
            pl.BlockSpec((_BN, 1), lambda i: (i, 0)),
        ],
        out_shape=[
            jax.ShapeDtypeStruct((N, NUM_CLASSES), jnp.float32),
            jax.ShapeDtypeStruct((N, 1), jnp.float32),
        ],
    )(u, stats, gamma.reshape(1, HIDDEN), beta.reshape(1, HIDDEN), Wcat, bcat)


def _chunk_major(h, nchunks):
    return jnp.transpose(h.reshape(N, nchunks, CHUNK), (1, 0, 2)).reshape(
        nchunks * N, CHUNK)


def kernel(x, edge_index, edge_attr, We1, be1, W1a, b1a, W1b, b1b, gamma1,
           beta1, We2, be2, W2a, b2a, W2b, b2b, gamma2, beta2, Wc, bc, Wr, br,
           logit_bias):
    src = edge_index[0]
    dst = edge_index[1]

    sd_t = jnp.pad((src * 65536 + dst).reshape(NS, EPT), ((0, 0), (0, EPAD)),
                   constant_values=N).reshape(NS * NSTEP, K)

    e1 = _edge_mm(edge_attr, We1, be1, nchunks=2)
    xflat1 = _chunk_major(x, 2)
    a1 = _sc_aggregate(xflat1, e1, sd_t, nchunks=2)
    u1, st1 = _mlp_stats(x, a1.reshape(2, N, CHUNK), W1a, b1a, W1b, b1b, 2)
    h1 = _bn_relu(u1, st1, gamma1, beta1)

    e2 = _edge_mm(edge_attr, We2, be2, nchunks=4)
    xflat2 = _chunk_major(h1, 4)
    a2 = _sc_aggregate(xflat2, e2, sd_t, nchunks=4)
    u2, st2 = _mlp_stats(h1, a2.reshape(4, N, CHUNK), W2a, b2a, W2b, b2b, 4)

    Wcat = jnp.concatenate([Wc, Wr], axis=1)
    bcat = jnp.concatenate([bc + logit_bias, br]).reshape(1, NUM_CLASSES + 1)
    logits, reg = _bn_relu_heads(u2, st2, gamma2, beta2, Wcat, bcat)
    return (logits, reg[:, 0])


# speedup vs baseline: 2.2301x; 1.0005x over previous
import functools

import jax
import jax.numpy as jnp
from jax import lax
from jax.experimental import pallas as pl
from jax.experimental.pallas import tpu as pltpu
from jax.experimental.pallas import tpu_sc as plsc

N = 10000
E = 160000
IN_DIM = 256
HIDDEN = 512
NUM_CLASSES = 4

NC = 2
NS = 16
LANES = 16
CHUNK = 128
K = 128
KH = 64
EPT = E // NS
NSTEP = pl.cdiv(EPT, K)
EPTP = NSTEP * K
EPAD = EPTP - EPT
AGGR_ROWS = N + 8
STRIPE = 640
ZCOP = 40


def _sc_aggregate(xflat, eflat, sds, nchunks):
    npass = nchunks // NC
    mesh = plsc.VectorSubcoreMesh(core_axis_name="c", subcore_axis_name="s")
    idx_rows = pl.cdiv(NSTEP, LANES) * LANES

    @functools.partial(
        pl.kernel,
        mesh=mesh,
        out_type=jax.ShapeDtypeStruct((nchunks * N, CHUNK), jnp.float32),
        scratch_types=[
            pltpu.VMEM((1, idx_rows), jnp.int32),
            pltpu.VMEM((idx_rows, K), jnp.int32),
            pltpu.VMEM((1, KH), jnp.int32),
            pltpu.VMEM((1, KH), jnp.int32),
            pltpu.VMEM((1, KH), jnp.int32),
            pltpu.VMEM((1, KH), jnp.int32),
            pltpu.VMEM((1, KH), jnp.int32),
            pltpu.VMEM((1, KH), jnp.int32),
            pltpu.VMEM((KH, CHUNK), jnp.float32),
            pltpu.VMEM((KH, CHUNK), jnp.float32),
            pltpu.VMEM((KH, CHUNK), jnp.float32),
            pltpu.VMEM((KH, CHUNK), jnp.float32),
            pltpu.VMEM_SHARED((AGGR_ROWS, CHUNK), jnp.float32),
            pltpu.SemaphoreType.DMA,
            pltpu.SemaphoreType.DMA,
        ],
    )
    def k(x_hbm, e_hbm, sds_hbm, out_hbm, ridx_v, sd_v,
          gsrc0, gsrc1, gdst0, gdst1, geid0, geid1,
          xb0, xb1, eb0, eb1, aggr, sem, sem_s):
        gsrc = (gsrc0, gsrc1)
        gdst = (gdst0, gdst1)
        geid = (geid0, geid1)
        xb = (xb0, xb1)
        eb = (eb0, eb1)
        c = lax.axis_index("c")
        s = lax.axis_index("s")
        row0 = s * STRIPE
        ncop = jnp.where(s == NS - 1, (N - (NS - 1) * STRIPE) // ZCOP,
                         STRIPE // ZCOP)
        lane = lax.iota(jnp.int32, LANES)
        ebase = s * EPT

        for j in range(idx_rows // LANES):
            ridx_v[0, pl.ds(j * LANES, LANES)] = jnp.minimum(
                lane + (s * NSTEP + j * LANES), NS * NSTEP - 1)

        pltpu.async_copy(sds_hbm.at[ridx_v.at[0]], sd_v, sem).wait()

        def build_lists(i, f, slot):
            t = jnp.minimum(lax.shift_right_logical(i, 1), NSTEP - 1)
            h = i - 2 * lax.shift_right_logical(i, 1)
            for j in range(KH // LANES):
                sl = pl.ds(j * LANES, LANES)
                v = sd_v[t, pl.ds(h * KH + j * LANES, LANES)]
                q = lax.shift_right_logical(v, 16)
                gsrc[slot][0, sl] = q + f * N
                gdst[slot][0, sl] = v - q * 65536
                geid[slot][0, sl] = f * E + jnp.minimum(
                    lane + (ebase + i * KH + j * LANES), E - 1)

        def issue(slot):
            pltpu.async_copy(e_hbm.at[geid[slot].at[0]], eb[slot], sem)
            pltpu.async_copy(x_hbm.at[gsrc[slot].at[0]], xb[slot], sem)

        def drain(slot):
            pltpu.make_async_copy(e_hbm.at[geid[slot].at[0]], eb[slot],
                                  sem).wait()
            pltpu.make_async_copy(x_hbm.at[gsrc[slot].at[0]], xb[slot],
                                  sem).wait()

        def drain_scatter(slot):
            pltpu.make_async_copy(xb[slot], aggr.at[gdst[slot].at[0]],
                                  sem_s).wait()

        def zxrow(r, carry):
            for j in range(CHUNK // LANES):
                xb0[r, pl.ds(j * LANES, LANES)] = jnp.zeros((LANES,),
                                                            jnp.float32)
            return carry

        def zerow(r, carry):
            for j in range(CHUNK // LANES):
                eb1[r, pl.ds(j * LANES, LANES)] = jnp.zeros((LANES,),
                                                            jnp.float32)
            return carry

        for p in range(npass):
            f = p * NC + c

            lax.fori_loop(0, ZCOP, zxrow, 0)
            lax.fori_loop(0, KH, zerow, 0)

            def zcopy(i, carry):
                pltpu.sync_copy(xb0.at[pl.ds(0, ZCOP)],
                                aggr.at[pl.ds(row0 + i * ZCOP, ZCOP)])
                return carry

            lax.fori_loop(0, ncop, zcopy, 0)
            plsc.subcore_barrier()

            build_lists(jnp.int32(0), f, 0)
            issue(0)
            for j in range(KH // LANES):
                gdst1[0, pl.ds(j * LANES, LANES)] = jnp.full(
                    (LANES,), N, jnp.int32)
            pltpu.async_copy(eb1, aggr.at[gdst1.at[0]], sem_s, add=True)

            def step(t, carry):
                for h in range(2):
                    i = 2 * t + h
                    b = h
                    nb = 1 - h
                    drain(b)
                    drain_scatter(nb)
                    build_lists(i + 1, f, nb)
                    issue(nb)

                    def crow(r, carry2):
                        for j in range(CHUNK // LANES):
                            sl = pl.ds(j * LANES, LANES)
                            xb[b][r, sl] = jnp.maximum(
                                xb[b][r, sl] + eb[b][r, sl], 0.0)
                        return carry2

                    lax.fori_loop(0, KH, crow, 0)
                    pltpu.async_copy(xb[b], aggr.at[gdst[b].at[0]], sem_s,
                                     add=True)
                return carry

            lax.fori_loop(0, NSTEP, step, 0)
            drain(0)
            drain_scatter(1)
            plsc.subcore_barrier()

            def ocopy(i, carry):
                r0 = row0 + i * ZCOP
                pltpu.sync_copy(aggr.at[pl.ds(r0, ZCOP)],
                                out_hbm.at[pl.ds(f * N + r0, ZCOP)])
                return carry

            lax.fori_loop(0, ncop, ocopy, 0)

    return k(xflat, eflat, sds)


_BE = 2000
_BN = 1000


def _edge_mm(edge_attr, We, be, nchunks):
    eb2 = be.reshape(nchunks, 1, CHUNK)

    def body(a_ref, w_ref, b_ref, o_ref):
        o_ref[...] = (
            jnp.dot(a_ref[...], w_ref[...], preferred_element_type=jnp.float32)
            + b_ref[0]
        )

    grid = (nchunks, E // _BE)
    return pl.pallas_call(
        body,
        grid=grid,
        in_specs=[
            pl.BlockSpec((_BE, edge_attr.shape[1]), lambda f, i: (i, 0)),
            pl.BlockSpec((edge_attr.shape[1], CHUNK), lambda f, i: (0, f)),
            pl.BlockSpec((1, 1, CHUNK), lambda f, i: (f, 0, 0)),
        ],
        out_specs=pl.BlockSpec((_BE, CHUNK), lambda f, i: (f * (E // _BE) + i, 0)),
        out_shape=jax.ShapeDtypeStruct((nchunks * E, CHUNK), jnp.float32),
    )(edge_attr, We, eb2)


def _mlp_stats(h, aggr, Wa, ba, Wb, bb, nchunks):
    in_dim = h.shape[1]

    def body(h_ref, a_ref, wa_ref, ba_ref, wb_ref, bb_ref, u_ref, st_ref):
        i = pl.program_id(0)
        hin = h_ref[...] + jnp.concatenate(
            [a_ref[j] for j in range(nchunks)], axis=-1)
        t = jnp.maximum(
            jnp.dot(hin, wa_ref[...], preferred_element_type=jnp.float32)
            + ba_ref[...], 0.0)
        u = jnp.dot(t, wb_ref[...], preferred_element_type=jnp.float32) + bb_ref[...]
        u_ref[...] = u

        @pl.when(i == 0)
        def _():
            st_ref[...] = jnp.zeros_like(st_ref)

        st_ref[0:1, :] += jnp.sum(u, axis=0, keepdims=True)
        st_ref[1:2, :] += jnp.sum(u * u, axis=0, keepdims=True)

    return pl.pallas_call(
        body,
        grid=(N // _BN,),
        in_specs=[
            pl.BlockSpec((_BN, in_dim), lambda i: (i, 0)),
            pl.BlockSpec((nchunks, _BN, CHUNK), lambda i: (0, i, 0)),
            pl.BlockSpec((in_dim, HIDDEN), lambda i: (0, 0)),
            pl.BlockSpec((1, HIDDEN), lambda i: (0, 0)),
            pl.BlockSpec((HIDDEN, HIDDEN), lambda i: (0, 0)),
            pl.BlockSpec((1, HIDDEN), lambda i: (0, 0)),
        ],
        out_specs=[
            pl.BlockSpec((_BN, HIDDEN), lambda i: (i, 0)),
            pl.BlockSpec((8, HIDDEN), lambda i: (0, 0)),
        ],
        out_shape=[
            jax.ShapeDtypeStruct((N, HIDDEN), jnp.float32),
            jax.ShapeDtypeStruct((8, HIDDEN), jnp.float32),
        ],
    )(h, aggr, Wa, ba.reshape(1, HIDDEN), Wb, bb.reshape(1, HIDDEN))


def _bn_relu(u, stats, gamma, beta):
    def body(u_ref, st_ref, g_ref, b_ref, o_ref):
        mean = st_ref[0:1, :] * (1.0 / N)
        var = st_ref[1:2, :] * (1.0 / N) - mean * mean
        rstd = lax.rsqrt(var + 1e-5)
        o_ref[...] = jnp.maximum(
            (u_ref[...] - mean) * rstd * g_ref[...] + b_ref[...], 0.0)

    return pl.pallas_call(
        body,
        grid=(N // _BN,),
        in_specs=[
            pl.BlockSpec((_BN, HIDDEN), lambda i: (i, 0)),
            pl.BlockSpec((8, HIDDEN), lambda i: (0, 0)),
            pl.BlockSpec((1, HIDDEN), lambda i: (0, 0)),
            pl.BlockSpec((1, HIDDEN), lambda i: (0, 0)),
        ],
        out_specs=pl.BlockSpec((_BN, HIDDEN), lambda i: (i, 0)),
        out_shape=jax.ShapeDtypeStruct((N, HIDDEN), jnp.float32),
    )(u, stats, gamma.reshape(1, HIDDEN), beta.reshape(1, HIDDEN))


def _bn_relu_heads(u, stats, gamma, beta, Wcat, bcat):
    nh = Wcat.shape[1]

    def body(u_ref, st_ref, g_ref, b_ref, w_ref, hb_ref, lo_ref, re_ref):
        mean = st_ref[0:1, :] * (1.0 / N)
        var = st_ref[1:2, :] * (1.0 / N) - mean * mean
        rstd = lax.rsqrt(var + 1e-5)
        h = jnp.maximum(
            (u_ref[...] - mean) * rstd * g_ref[...] + b_ref[...], 0.0)
        o = jnp.dot(h, w_ref[...], preferred_element_type=jnp.float32) + hb_ref[...]
        lo_ref[...] = o[:, :NUM_CLASSES]
        re_ref[...] = o[:, NUM_CLASSES:NUM_CLASSES + 1]

    return pl.pallas_call(
        body,
        grid=(N // _BN,),
        in_specs=[
            pl.BlockSpec((_BN, HIDDEN), lambda i: (i, 0)),
            pl.BlockSpec((8, HIDDEN), lambda i: (0, 0)),
            pl.BlockSpec((1, HIDDEN), lambda i: (0, 0)),
            pl.BlockSpec((1, HIDDEN), lambda i: (0, 0)),
            pl.BlockSpec((HIDDEN, nh), lambda i: (0, 0)),
            pl.BlockSpec((1, nh), lambda i: (0, 0)),
        ],
        out_specs=[
            pl.BlockSpec((_BN, NUM_CLASSES), lambda i: (i, 0)),
            pl.BlockSpec((_BN, 1), lambda i: (i, 0)),
        ],
        out_shape=[
            jax.ShapeDtypeStruct((N, NUM_CLASSES), jnp.float32),
            jax.ShapeDtypeStruct((N, 1), jnp.float32),
        ],
    )(u, stats, gamma.reshape(1, HIDDEN), beta.reshape(1, HIDDEN), Wcat, bcat)


def _chunk_major(h, nchunks):
    return jnp.transpose(h.reshape(N, nchunks, CHUNK), (1, 0, 2)).reshape(
        nchunks * N, CHUNK)


def kernel(x, edge_index, edge_attr, We1, be1, W1a, b1a, W1b, b1b, gamma1,
           beta1, We2, be2, W2a, b2a, W2b, b2b, gamma2, beta2, Wc, bc, Wr, br,
           logit_bias):
    src = edge_index[0]
    dst = edge_index[1]

    sd_t = jnp.pad((src * 65536 + dst).reshape(NS, EPT), ((0, 0), (0, EPAD)),
                   constant_values=N).reshape(NS * NSTEP, K)

    e1 = _edge_mm(edge_attr, We1, be1, nchunks=2)
    xflat1 = _chunk_major(x, 2)
    a1 = _sc_aggregate(xflat1, e1, sd_t, nchunks=2)
    u1, st1 = _mlp_stats(x, a1.reshape(2, N, CHUNK), W1a, b1a, W1b, b1b, 2)
    h1 = _bn_relu(u1, st1, gamma1, beta1)

    e2 = _edge_mm(edge_attr, We2, be2, nchunks=4)
    xflat2 = _chunk_major(h1, 4)
    a2 = _sc_aggregate(xflat2, e2, sd_t, nchunks=4)
    u2, st2 = _mlp_stats(h1, a2.reshape(4, N, CHUNK), W2a, b2a, W2b, b2b, 4)

    Wcat = jnp.concatenate([Wc, Wr], axis=1)
    bcat = jnp.concatenate([bc + logit_bias, br]).reshape(1, NUM_CLASSES + 1)
    logits, reg = _bn_relu_heads(u2, st2, gamma2, beta2, Wcat, bcat)
    return (logits, reg[:, 0])
